# baseline (speedup 1.0000x reference)
import functools

import jax
import jax.numpy as jnp
from jax import lax
from jax.experimental import pallas as pl
from jax.experimental.pallas import tpu as pltpu
from jax.experimental.pallas import tpu_sc as plsc

SHIFT = -16
NEG_INF = float("-inf")
LANES = 16
NWIN = 7


def _window_bounds(leff):
    bounds = [(jnp.int32(0), leff)]
    for lvl in (1, 2):
        d = 2 ** lvl
        k = (leff + d - 1) // d
        s = leff // d
        for j in range(d):
            lo = jnp.int32(j) * s
            hi = jnp.minimum(lo + k, leff)
            bounds.append((lo, hi))
    return bounds


def _tree_max(terms):
    while len(terms) > 1:
        nxt = [jnp.maximum(a, b) for a, b in zip(terms[::2], terms[1::2])]
        if len(terms) % 2:
            nxt.append(terms[-1])
        terms = nxt
    return terms[0]


def _tc_body(lens_ref, x_ref, o1_ref, o2_ref, o3_ref):
    i = pl.program_id(0)
    t = pl.program_id(1)
    T = x_ref.shape[2]
    L = pl.num_programs(1) * T

    leff = jnp.minimum(lens_ref[i] - SHIFT, L)
    xb = x_ref[0]
    pos = lax.broadcasted_iota(jnp.int32, (1, T), 1) + t * T

    @pl.when(t == 0)
    def _():
        o1_ref[...] = jnp.full_like(o1_ref, NEG_INF)
        o2_ref[...] = jnp.full_like(o2_ref, NEG_INF)
        o3_ref[...] = jnp.full_like(o3_ref, NEG_INF)

    maxes = []
    for lo, hi in _window_bounds(leff):
        mask = (pos >= lo) & (pos < hi)
        maxes.append(jnp.max(jnp.where(mask, xb, NEG_INF), axis=1))

    o1_ref[0, 0, :] = jnp.maximum(o1_ref[0, 0, :], maxes[0])
    for j in range(2):
        o2_ref[0, :, j] = jnp.maximum(o2_ref[0, :, j], maxes[1 + j])
    for j in range(4):
        o3_ref[0, :, j] = jnp.maximum(o3_ref[0, :, j], maxes[3 + j])


def _tc_call(x, lens):
    B, C, L = x.shape
    T = 1024
    NT = L // T
    grid_spec = pltpu.PrefetchScalarGridSpec(
        num_scalar_prefetch=1,
        grid=(B, NT),
        in_specs=[
            pl.BlockSpec((1, C, T), lambda i, t, lens: (i, 0, t)),
        ],
        out_specs=[
            pl.BlockSpec((1, 1, C), lambda i, t, lens: (i, 0, 0)),
            pl.BlockSpec((1, C, 2), lambda i, t, lens: (i, 0, 0)),
            pl.BlockSpec((1, C, 4), lambda i, t, lens: (i, 0, 0)),
        ],
    )
    o1, o2, o3 = pl.pallas_call(
        _tc_body,
        grid_spec=grid_spec,
        out_shape=[
            jax.ShapeDtypeStruct((B, 1, C), jnp.float32),
            jax.ShapeDtypeStruct((B, C, 2), jnp.float32),
            jax.ShapeDtypeStruct((B, C, 4), jnp.float32),
        ],
    )(lens, x)
    return jnp.concatenate(
        [o1.reshape(B, C), o2.reshape(B, 2 * C), o3.reshape(B, 4 * C)],
        axis=1)


def _make_sc_kernel(B, C, L, interpret=False):
    ROWS = B * C
    NW = 32
    RPW = ROWS // NW
    NSPAN = L // 256
    NBLK = L // 64
    NC = 2
    SB = 8
    NSB = RPW // SB

    def body(x_hbm, lens_hbm, out_hbm, lens_v, sbuf0, sbuf1, sv_ref, sv64_ref,
             s64_ref, pi_ref, ix_ref, p64_ref, praw_ref, corr_ref, scr_ref,
             out_buf, sem0, sem1):
        wid = lax.axis_index("s") * NC + lax.axis_index("c")
        row0 = wid * RPW
        sample = row0 // C

        pltpu.sync_copy(lens_hbm, lens_v)
        iota = lax.broadcasted_iota(jnp.int32, (LANES,), 0)
        iota16x = iota * LANES
        lvec = plsc.load_gather(lens_v, [iota * 0 + sample])
        leff = jnp.minimum(lvec[0] - SHIFT, L)
        windows = _window_bounds(leff)
        k2 = (leff + 1) // 2

        def vpen(cond):
            return jnp.where(cond, 0.0, NEG_INF).astype(jnp.float32)

        span_base = iota * 256
        raw_bases = {}
        for w in (2, 3, 4, 5, 6):
            lo, hi = windows[w]
            pi_ref[pl.ds(w * LANES, LANES)] = vpen(
                (span_base >= lo) & (span_base + 256 <= hi))
            b0blk = (lo // 256) * 4
            b1blk = ((hi - 1) // 256) * 4
            idx = jnp.where(iota < 4, b0blk + iota,
                            jnp.where(iota < 8, b1blk + (iota - 4), 0))
            ix_ref[pl.ds(w * LANES, LANES)] = idx
            bvec = idx * 64
            p64_ref[pl.ds(w * LANES, LANES)] = vpen(
                (bvec >= lo) & (bvec + 64 <= hi))
            bb = ((lo // 64) * 64, ((hi - 1) // 64) * 64)
            raw_bases[w] = bb
            for e in range(2):
                for k in range(4):
                    pos = iota + bb[e] + k * LANES
                    praw_ref[pl.ds(((w * 2 + e) * 4 + k) * LANES, LANES)] = \
                        vpen((pos >= lo) & (pos < hi))
        e1 = k2 - 1
        cbase = (e1 // LANES) * LANES
        corr_ref[...] = vpen(iota + cbase == e1)

        def compute_row(buf, b0, slot):
            for g in range(NSPAN):
                subs = []
                for q in range(4):
                    base = b0 + g * 256 + q * 64
                    vs = [buf[pl.ds(base + k * LANES, LANES)]
                          for k in range(4)]
                    m = _tree_max(vs)
                    sv64_ref[pl.ds((g * 4 + q) * LANES, LANES)] = m
                    subs.append(m)
                sv_ref[pl.ds(g * LANES, LANES)] = _tree_max(subs)

            sspan = _tree_max([plsc.load_gather(sv_ref, [iota16x + j])
                               for j in range(LANES)])
            for p in range(4):
                part = _tree_max(
                    [plsc.load_gather(sv64_ref, [p * 256 + iota16x + j])
                     for j in range(LANES)])
                s64_ref[pl.ds(p * LANES, LANES)] = part

            accs = [None] * NWIN
            for w in (2, 3, 4, 5, 6):
                idx = ix_ref[pl.ds(w * LANES, LANES)]
                terms = [
                    sspan + pi_ref[pl.ds(w * LANES, LANES)],
                    plsc.load_gather(s64_ref, [idx])
                    + p64_ref[pl.ds(w * LANES, LANES)],
                ]
                for e in range(2):
                    for k in range(4):
                        terms.append(
                            buf[pl.ds(b0 + raw_bases[w][e] + k * LANES, LANES)]
                            + praw_ref[pl.ds(
                                ((w * 2 + e) * 4 + k) * LANES, LANES)])
                accs[w] = _tree_max(terms)

            corr = buf[pl.ds(b0 + cbase, LANES)] + corr_ref[...]
            accs[1] = jnp.maximum(jnp.maximum(accs[3], accs[4]), corr)
            accs[0] = jnp.maximum(accs[1], accs[2])

            for w in range(NWIN):
                scr_ref[pl.ds(w * LANES, LANES)] = accs[w]
            gathered = [plsc.load_gather(scr_ref, [iota16x + j])
                        for j in range(LANES)]
            out_buf[pl.ds(slot * LANES, LANES)] = _tree_max(gathered)

        def sb_src(sbi):
            return x_hbm.at[pl.ds((row0 + sbi * SB) * L, SB * L)]

        pltpu.async_copy(sb_src(0), sbuf0, sem0)
        pltpu.async_copy(sb_src(1), sbuf1, sem1)

        @pl.loop(0, NSB, step=2)
        def sb_loop(g):
            for b, (sbuf, sem) in enumerate(((sbuf0, sem0), (sbuf1, sem1))):
                sbi = g + b
                pltpu.make_async_copy(sb_src(0), sbuf, sem).wait()

                @pl.loop(0, SB)
                def row_loop(r):
                    compute_row(sbuf, r * L, sbi * SB + r)

                nxt = sbi + 2

                @pl.when(nxt < NSB)
                def _():
                    pltpu.async_copy(sb_src(nxt), sbuf, sem)

        pltpu.sync_copy(out_buf, out_hbm.at[pl.ds(row0 * LANES, RPW * LANES)])

    return pl.kernel(
        body,
        out_type=jax.ShapeDtypeStruct((ROWS * LANES,), jnp.float32),
        mesh=plsc.VectorSubcoreMesh(core_axis_name="c", subcore_axis_name="s",
                                    num_cores=NC, num_subcores=16),
        compiler_params=pltpu.CompilerParams(needs_layout_passes=False),
        scratch_types=[
            pltpu.VMEM((LANES,), jnp.int32),
            pltpu.VMEM((SB * L,), jnp.float32),
            pltpu.VMEM((SB * L,), jnp.float32),
            pltpu.VMEM((NSPAN * LANES,), jnp.float32),
            pltpu.VMEM((NBLK * LANES,), jnp.float32),
            pltpu.VMEM((NBLK,), jnp.float32),
            pltpu.VMEM((NWIN * LANES,), jnp.float32),
            pltpu.VMEM((NWIN * LANES,), jnp.int32),
            pltpu.VMEM((NWIN * LANES,), jnp.float32),
            pltpu.VMEM((NWIN * 8 * LANES,), jnp.float32),
            pltpu.VMEM((LANES,), jnp.float32),
            pltpu.VMEM((LANES * LANES,), jnp.float32),
            pltpu.VMEM((RPW * LANES,), jnp.float32),
            pltpu.SemaphoreType.DMA,
            pltpu.SemaphoreType.DMA,
        ],
        interpret=interpret,
    )


def kernel(x, orig_len):
    B, C, L = x.shape
    BSC = 4
    lens = jnp.asarray(orig_len, jnp.int32)
    sc = _make_sc_kernel(BSC, C, L)
    out = sc(x[:BSC].reshape(BSC * C * L), lens).reshape(BSC, C, LANES)
    sc_rows = jnp.concatenate(
        [out[:, :, 0],
         out[:, :, 1:3].reshape(BSC, 2 * C),
         out[:, :, 3:7].reshape(BSC, 4 * C)], axis=1)
    tc_rows = _tc_call(x[BSC:], lens[BSC:])
    return jnp.concatenate([sc_rows, tc_rows], axis=0)


# speedup vs baseline: 1.1130x; 1.1130x over previous
import functools

import jax
import jax.numpy as jnp
from jax import lax
from jax.experimental import pallas as pl
from jax.experimental.pallas import tpu as pltpu
from jax.experimental.pallas import tpu_sc as plsc

SHIFT = -16
NEG_INF = float("-inf")
LANES = 16
NWIN = 7


def _window_bounds(leff):
    bounds = [(jnp.int32(0), leff)]
    for lvl in (1, 2):
        d = 2 ** lvl
        k = (leff + d - 1) // d
        s = leff // d
        for j in range(d):
            lo = jnp.int32(j) * s
            hi = jnp.minimum(lo + k, leff)
            bounds.append((lo, hi))
    return bounds


def _tree_max(terms):
    while len(terms) > 1:
        nxt = [jnp.maximum(a, b) for a, b in zip(terms[::2], terms[1::2])]
        if len(terms) % 2:
            nxt.append(terms[-1])
        terms = nxt
    return terms[0]


def _tc_body(lens_ref, x_ref, o1_ref, o2_ref, o3_ref, *, b0):
    i = pl.program_id(0)
    t = pl.program_id(1)
    T = x_ref.shape[2]
    L = pl.num_programs(1) * T

    leff = jnp.minimum(lens_ref[i + b0] - SHIFT, L)
    xb = x_ref[0]
    pos = lax.broadcasted_iota(jnp.int32, (1, T), 1) + t * T

    @pl.when(t == 0)
    def _():
        o1_ref[...] = jnp.full_like(o1_ref, NEG_INF)
        o2_ref[...] = jnp.full_like(o2_ref, NEG_INF)
        o3_ref[...] = jnp.full_like(o3_ref, NEG_INF)

    maxes = []
    for lo, hi in _window_bounds(leff):
        mask = (pos >= lo) & (pos < hi)
        maxes.append(jnp.max(jnp.where(mask, xb, NEG_INF), axis=1))

    o1_ref[0, 0, :] = jnp.maximum(o1_ref[0, 0, :], maxes[0])
    for j in range(2):
        o2_ref[0, :, j] = jnp.maximum(o2_ref[0, :, j], maxes[1 + j])
    for j in range(4):
        o3_ref[0, :, j] = jnp.maximum(o3_ref[0, :, j], maxes[3 + j])


def _tc_call(x, lens, b0):
    B, C, L = x.shape
    NB = B - b0
    T = 1024
    NT = L // T
    grid_spec = pltpu.PrefetchScalarGridSpec(
        num_scalar_prefetch=1,
        grid=(NB, NT),
        in_specs=[
            pl.BlockSpec((1, C, T), lambda i, t, lens: (i + b0, 0, t)),
        ],
        out_specs=[
            pl.BlockSpec((1, 1, C), lambda i, t, lens: (i, 0, 0)),
            pl.BlockSpec((1, C, 2), lambda i, t, lens: (i, 0, 0)),
            pl.BlockSpec((1, C, 4), lambda i, t, lens: (i, 0, 0)),
        ],
    )
    o1, o2, o3 = pl.pallas_call(
        functools.partial(_tc_body, b0=b0),
        grid_spec=grid_spec,
        out_shape=[
            jax.ShapeDtypeStruct((NB, 1, C), jnp.float32),
            jax.ShapeDtypeStruct((NB, C, 2), jnp.float32),
            jax.ShapeDtypeStruct((NB, C, 4), jnp.float32),
        ],
    )(lens, x)
    return jnp.concatenate(
        [o1.reshape(NB, C), o2.reshape(NB, 2 * C), o3.reshape(NB, 4 * C)],
        axis=1)


def _make_sc_kernel(B, C, L, interpret=False):
    ROWS = B * C
    NW = 32
    RPW = ROWS // NW
    NSPAN = L // 256
    NBLK = L // 64
    NC = 2
    SB = 8
    NSB = RPW // SB

    def body(x_hbm, lens_hbm, out_hbm, lens_v, sbuf0, sbuf1, sv_ref, sv64_ref,
             s64_ref, pi_ref, ix_ref, p64_ref, praw_ref, corr_ref, scr_ref,
             out_buf, sem0, sem1):
        wid = lax.axis_index("s") * NC + lax.axis_index("c")
        row0 = wid * RPW
        sample = row0 // C

        pltpu.sync_copy(lens_hbm, lens_v)
        iota = lax.broadcasted_iota(jnp.int32, (LANES,), 0)
        iota16x = iota * LANES
        lvec = plsc.load_gather(lens_v, [iota * 0 + sample])
        leff = jnp.minimum(lvec[0] - SHIFT, L)
        windows = _window_bounds(leff)
        k2 = (leff + 1) // 2

        def vpen(cond):
            return jnp.where(cond, 0.0, NEG_INF).astype(jnp.float32)

        span_base = iota * 256
        raw_bases = {}
        for w in (2, 3, 4, 5, 6):
            lo, hi = windows[w]
            pi_ref[pl.ds(w * LANES, LANES)] = vpen(
                (span_base >= lo) & (span_base + 256 <= hi))
            b0blk = (lo // 256) * 4
            b1blk = ((hi - 1) // 256) * 4
            idx = jnp.where(iota < 4, b0blk + iota,
                            jnp.where(iota < 8, b1blk + (iota - 4), 0))
            ix_ref[pl.ds(w * LANES, LANES)] = idx
            bvec = idx * 64
            p64_ref[pl.ds(w * LANES, LANES)] = vpen(
                (bvec >= lo) & (bvec + 64 <= hi))
            bb = ((lo // 64) * 64, ((hi - 1) // 64) * 64)
            raw_bases[w] = bb
            for e in range(2):
                for k in range(4):
                    pos = iota + bb[e] + k * LANES
                    praw_ref[pl.ds(((w * 2 + e) * 4 + k) * LANES, LANES)] = \
                        vpen((pos >= lo) & (pos < hi))
        e1 = k2 - 1
        cbase = (e1 // LANES) * LANES
        corr_ref[...] = vpen(iota + cbase == e1)

        def compute_row(buf, b0, slot):
            for g in range(NSPAN):
                subs = []
                for q in range(4):
                    base = b0 + g * 256 + q * 64
                    vs = [buf[pl.ds(base + k * LANES, LANES)]
                          for k in range(4)]
                    m = _tree_max(vs)
                    sv64_ref[pl.ds((g * 4 + q) * LANES, LANES)] = m
                    subs.append(m)
                sv_ref[pl.ds(g * LANES, LANES)] = _tree_max(subs)

            sspan = _tree_max([plsc.load_gather(sv_ref, [iota16x + j])
                               for j in range(LANES)])
            for p in range(4):
                part = _tree_max(
                    [plsc.load_gather(sv64_ref, [p * 256 + iota16x + j])
                     for j in range(LANES)])
                s64_ref[pl.ds(p * LANES, LANES)] = part

            accs = [None] * NWIN
            for w in (2, 3, 4, 5, 6):
                idx = ix_ref[pl.ds(w * LANES, LANES)]
                terms = [
                    sspan + pi_ref[pl.ds(w * LANES, LANES)],
                    plsc.load_gather(s64_ref, [idx])
                    + p64_ref[pl.ds(w * LANES, LANES)],
                ]
                for e in range(2):
                    for k in range(4):
                        terms.append(
                            buf[pl.ds(b0 + raw_bases[w][e] + k * LANES, LANES)]
                            + praw_ref[pl.ds(
                                ((w * 2 + e) * 4 + k) * LANES, LANES)])
                accs[w] = _tree_max(terms)

            corr = buf[pl.ds(b0 + cbase, LANES)] + corr_ref[...]
            accs[1] = jnp.maximum(jnp.maximum(accs[3], accs[4]), corr)
            accs[0] = jnp.maximum(accs[1], accs[2])

            for w in range(NWIN):
                scr_ref[pl.ds(w * LANES, LANES)] = accs[w]
            gathered = [plsc.load_gather(scr_ref, [iota16x + j])
                        for j in range(LANES)]
            out_buf[pl.ds(slot * LANES, LANES)] = _tree_max(gathered)

        def sb_src(sbi):
            return x_hbm.at[pl.ds((row0 + sbi * SB) * L, SB * L)]

        pltpu.async_copy(sb_src(0), sbuf0, sem0)
        pltpu.async_copy(sb_src(1), sbuf1, sem1)

        @pl.loop(0, NSB, step=2)
        def sb_loop(g):
            for b, (sbuf, sem) in enumerate(((sbuf0, sem0), (sbuf1, sem1))):
                sbi = g + b
                pltpu.make_async_copy(sb_src(0), sbuf, sem).wait()

                @pl.loop(0, SB)
                def row_loop(r):
                    compute_row(sbuf, r * L, sbi * SB + r)

                nxt = sbi + 2

                @pl.when(nxt < NSB)
                def _():
                    pltpu.async_copy(sb_src(nxt), sbuf, sem)

        pltpu.sync_copy(out_buf, out_hbm.at[pl.ds(row0 * LANES, RPW * LANES)])

    return pl.kernel(
        body,
        out_type=jax.ShapeDtypeStruct((ROWS * LANES,), jnp.float32),
        mesh=plsc.VectorSubcoreMesh(core_axis_name="c", subcore_axis_name="s",
                                    num_cores=NC, num_subcores=16),
        compiler_params=pltpu.CompilerParams(needs_layout_passes=False),
        scratch_types=[
            pltpu.VMEM((LANES,), jnp.int32),
            pltpu.VMEM((SB * L,), jnp.float32),
            pltpu.VMEM((SB * L,), jnp.float32),
            pltpu.VMEM((NSPAN * LANES,), jnp.float32),
            pltpu.VMEM((NBLK * LANES,), jnp.float32),
            pltpu.VMEM((NBLK,), jnp.float32),
            pltpu.VMEM((NWIN * LANES,), jnp.float32),
            pltpu.VMEM((NWIN * LANES,), jnp.int32),
            pltpu.VMEM((NWIN * LANES,), jnp.float32),
            pltpu.VMEM((NWIN * 8 * LANES,), jnp.float32),
            pltpu.VMEM((LANES,), jnp.float32),
            pltpu.VMEM((LANES * LANES,), jnp.float32),
            pltpu.VMEM((RPW * LANES,), jnp.float32),
            pltpu.SemaphoreType.DMA,
            pltpu.SemaphoreType.DMA,
        ],
        interpret=interpret,
    )


def kernel(x, orig_len):
    B, C, L = x.shape
    BSC = 4
    lens = jnp.asarray(orig_len, jnp.int32)
    sc = _make_sc_kernel(BSC, C, L)
    out = sc(x.reshape(B * C * L), lens).reshape(BSC, C, LANES)
    sc_rows = jnp.concatenate(
        [out[:, :, 0],
         out[:, :, 1:3].reshape(BSC, 2 * C),
         out[:, :, 3:7].reshape(BSC, 4 * C)], axis=1)
    tc_rows = _tc_call(x, lens, BSC)
    return jnp.concatenate([sc_rows, tc_rows], axis=0)


# speedup vs baseline: 2.0630x; 1.8536x over previous
import functools

import jax
import jax.numpy as jnp
from jax import lax
from jax.experimental import pallas as pl
from jax.experimental.pallas import tpu as pltpu
from jax.experimental.pallas import tpu_sc as plsc

SHIFT = -16
NEG_INF = float("-inf")
LANES = 16
NWIN = 7


def _window_bounds(leff):
    bounds = [(jnp.int32(0), leff)]
    for lvl in (1, 2):
        d = 2 ** lvl
        k = (leff + d - 1) // d
        s = leff // d
        for j in range(d):
            lo = jnp.int32(j) * s
            hi = jnp.minimum(lo + k, leff)
            bounds.append((lo, hi))
    return bounds


def _tree_max(terms):
    while len(terms) > 1:
        nxt = [jnp.maximum(a, b) for a, b in zip(terms[::2], terms[1::2])]
        if len(terms) % 2:
            nxt.append(terms[-1])
        terms = nxt
    return terms[0]


def _tc_body(lens_ref, x_ref, o1_ref, o2_ref, o3_ref, *, b0):
    i = pl.program_id(0)
    t = pl.program_id(1)
    T = x_ref.shape[2]
    L = pl.num_programs(1) * T

    leff = jnp.minimum(lens_ref[i + b0] - SHIFT, L)
    xb = x_ref[0]
    pos = lax.broadcasted_iota(jnp.int32, (1, T), 1) + t * T

    @pl.when(t == 0)
    def _():
        o1_ref[...] = jnp.full_like(o1_ref, NEG_INF)
        o2_ref[...] = jnp.full_like(o2_ref, NEG_INF)
        o3_ref[...] = jnp.full_like(o3_ref, NEG_INF)

    maxes = []
    for lo, hi in _window_bounds(leff):
        mask = (pos >= lo) & (pos < hi)
        maxes.append(jnp.max(jnp.where(mask, xb, NEG_INF), axis=1))

    o1_ref[0, 0, :] = jnp.maximum(o1_ref[0, 0, :], maxes[0])
    for j in range(2):
        o2_ref[0, :, j] = jnp.maximum(o2_ref[0, :, j], maxes[1 + j])
    for j in range(4):
        o3_ref[0, :, j] = jnp.maximum(o3_ref[0, :, j], maxes[3 + j])


def _tc_call(x, lens, b0):
    B, C, L = x.shape
    NB = B - b0
    T = 1024
    NT = L // T
    grid_spec = pltpu.PrefetchScalarGridSpec(
        num_scalar_prefetch=1,
        grid=(NB, NT),
        in_specs=[
            pl.BlockSpec((1, C, T), lambda i, t, lens: (i + b0, 0, t)),
        ],
        out_specs=[
            pl.BlockSpec((1, 1, C), lambda i, t, lens: (i, 0, 0)),
            pl.BlockSpec((1, C, 2), lambda i, t, lens: (i, 0, 0)),
            pl.BlockSpec((1, C, 4), lambda i, t, lens: (i, 0, 0)),
        ],
    )
    o1, o2, o3 = pl.pallas_call(
        functools.partial(_tc_body, b0=b0),
        grid_spec=grid_spec,
        out_shape=[
            jax.ShapeDtypeStruct((NB, 1, C), jnp.float32),
            jax.ShapeDtypeStruct((NB, C, 2), jnp.float32),
            jax.ShapeDtypeStruct((NB, C, 4), jnp.float32),
        ],
    )(lens, x)
    return jnp.concatenate(
        [o1.reshape(NB, C), o2.reshape(NB, 2 * C), o3.reshape(NB, 4 * C)],
        axis=1)


def _make_sc_kernel(B, C, L, interpret=False):
    ROWS = B * C
    NW = 32
    RPW = ROWS // NW
    NSPAN = L // 256
    NBLK = L // 64
    NC = 2
    SB = 8
    NSB = RPW // SB

    def body(x_hbm, lens_hbm, out_hbm, lens_v, sbuf0, sbuf1, sv_ref, sv64_ref,
             s64_ref, pi_ref, ix_ref, p64_ref, praw_ref, corr_ref, scr_ref,
             out_buf, sem0, sem1):
        wid = lax.axis_index("s") * NC + lax.axis_index("c")
        row0 = wid * RPW
        sample = row0 // C

        pltpu.sync_copy(lens_hbm, lens_v)
        iota = lax.broadcasted_iota(jnp.int32, (LANES,), 0)
        iota16x = iota * LANES
        lvec = plsc.load_gather(lens_v, [iota * 0 + sample])
        leff = jnp.minimum(lvec[0] - SHIFT, L)
        windows = _window_bounds(leff)
        k2 = (leff + 1) // 2

        def vpen(cond):
            return jnp.where(cond, 0.0, NEG_INF).astype(jnp.float32)

        span_base = iota * 256
        raw_bases = {}
        for w in (2, 3, 4, 5, 6):
            lo, hi = windows[w]
            pi_ref[pl.ds(w * LANES, LANES)] = vpen(
                (span_base >= lo) & (span_base + 256 <= hi))
            b0blk = (lo // 256) * 4
            b1blk = ((hi - 1) // 256) * 4
            idx = jnp.where(iota < 4, b0blk + iota,
                            jnp.where(iota < 8, b1blk + (iota - 4), 0))
            ix_ref[pl.ds(w * LANES, LANES)] = idx
            bvec = idx * 64
            p64_ref[pl.ds(w * LANES, LANES)] = vpen(
                (bvec >= lo) & (bvec + 64 <= hi))
            bb = ((lo // 64) * 64, ((hi - 1) // 64) * 64)
            raw_bases[w] = bb
            for e in range(2):
                for k in range(4):
                    pos = iota + bb[e] + k * LANES
                    praw_ref[pl.ds(((w * 2 + e) * 4 + k) * LANES, LANES)] = \
                        vpen((pos >= lo) & (pos < hi))
        e1 = k2 - 1
        cbase = (e1 // LANES) * LANES
        corr_ref[...] = vpen(iota + cbase == e1)

        def compute_row(buf, r, slot):
            for g in range(NSPAN):
                subs = []
                for q in range(4):
                    base = g * 256 + q * 64
                    vs = [buf[r, pl.ds(base + k * LANES, LANES)]
                          for k in range(4)]
                    m = _tree_max(vs)
                    sv64_ref[pl.ds((g * 4 + q) * LANES, LANES)] = m
                    subs.append(m)
                sv_ref[pl.ds(g * LANES, LANES)] = _tree_max(subs)

            sspan = _tree_max([plsc.load_gather(sv_ref, [iota16x + j])
                               for j in range(LANES)])
            for p in range(4):
                part = _tree_max(
                    [plsc.load_gather(sv64_ref, [p * 256 + iota16x + j])
                     for j in range(LANES)])
                s64_ref[pl.ds(p * LANES, LANES)] = part

            accs = [None] * NWIN
            for w in (2, 3, 4, 5, 6):
                idx = ix_ref[pl.ds(w * LANES, LANES)]
                terms = [
                    sspan + pi_ref[pl.ds(w * LANES, LANES)],
                    plsc.load_gather(s64_ref, [idx])
                    + p64_ref[pl.ds(w * LANES, LANES)],
                ]
                for e in range(2):
                    for k in range(4):
                        terms.append(
                            buf[r, pl.ds(raw_bases[w][e] + k * LANES, LANES)]
                            + praw_ref[pl.ds(
                                ((w * 2 + e) * 4 + k) * LANES, LANES)])
                accs[w] = _tree_max(terms)

            corr = buf[r, pl.ds(cbase, LANES)] + corr_ref[...]
            accs[1] = jnp.maximum(jnp.maximum(accs[3], accs[4]), corr)
            accs[0] = jnp.maximum(accs[1], accs[2])

            for w in range(NWIN):
                scr_ref[pl.ds(w * LANES, LANES)] = accs[w]
            gathered = [plsc.load_gather(scr_ref, [iota16x + j])
                        for j in range(LANES)]
            out_buf[pl.ds(slot * LANES, LANES)] = _tree_max(gathered)

        def sb_src(sbi):
            return x_hbm.at[pl.ds(row0 + sbi * SB, SB)]

        pltpu.async_copy(sb_src(0), sbuf0, sem0)
        pltpu.async_copy(sb_src(1), sbuf1, sem1)

        @pl.loop(0, NSB, step=2)
        def sb_loop(g):
            for b, (sbuf, sem) in enumerate(((sbuf0, sem0), (sbuf1, sem1))):
                sbi = g + b
                pltpu.make_async_copy(sb_src(0), sbuf, sem).wait()

                @pl.loop(0, SB)
                def row_loop(r):
                    compute_row(sbuf, r, sbi * SB + r)

                nxt = sbi + 2

                @pl.when(nxt < NSB)
                def _():
                    pltpu.async_copy(sb_src(nxt), sbuf, sem)

        pltpu.sync_copy(out_buf, out_hbm.at[pl.ds(row0 * LANES, RPW * LANES)])

    return pl.kernel(
        body,
        out_type=jax.ShapeDtypeStruct((ROWS * LANES,), jnp.float32),
        mesh=plsc.VectorSubcoreMesh(core_axis_name="c", subcore_axis_name="s",
                                    num_cores=NC, num_subcores=16),
        compiler_params=pltpu.CompilerParams(needs_layout_passes=False),
        scratch_types=[
            pltpu.VMEM((LANES,), jnp.int32),
            pltpu.VMEM((SB, L), jnp.float32),
            pltpu.VMEM((SB, L), jnp.float32),
            pltpu.VMEM((NSPAN * LANES,), jnp.float32),
            pltpu.VMEM((NBLK * LANES,), jnp.float32),
            pltpu.VMEM((NBLK,), jnp.float32),
            pltpu.VMEM((NWIN * LANES,), jnp.float32),
            pltpu.VMEM((NWIN * LANES,), jnp.int32),
            pltpu.VMEM((NWIN * LANES,), jnp.float32),
            pltpu.VMEM((NWIN * 8 * LANES,), jnp.float32),
            pltpu.VMEM((LANES,), jnp.float32),
            pltpu.VMEM((LANES * LANES,), jnp.float32),
            pltpu.VMEM((RPW * LANES,), jnp.float32),
            pltpu.SemaphoreType.DMA,
            pltpu.SemaphoreType.DMA,
        ],
        interpret=interpret,
    )


def kernel(x, orig_len):
    B, C, L = x.shape
    BSC = 4
    lens = jnp.asarray(orig_len, jnp.int32)
    sc = _make_sc_kernel(BSC, C, L)
    out = sc(x.reshape(B * C, L), lens).reshape(BSC, C, LANES)
    sc_rows = jnp.concatenate(
        [out[:, :, 0],
         out[:, :, 1:3].reshape(BSC, 2 * C),
         out[:, :, 3:7].reshape(BSC, 4 * C)], axis=1)
    tc_rows = _tc_call(x, lens, BSC)
    return jnp.concatenate([sc_rows, tc_rows], axis=0)


# speedup vs baseline: 2.1705x; 1.0521x over previous
import functools

import jax
import jax.numpy as jnp
from jax import lax
from jax.experimental import pallas as pl
from jax.experimental.pallas import tpu as pltpu
from jax.experimental.pallas import tpu_sc as plsc

SHIFT = -16
NEG_INF = float("-inf")
LANES = 16
NWIN = 7


def _window_bounds(leff):
    bounds = [(jnp.int32(0), leff)]
    for lvl in (1, 2):
        d = 2 ** lvl
        k = (leff + d - 1) // d
        s = leff // d
        for j in range(d):
            lo = jnp.int32(j) * s
            hi = jnp.minimum(lo + k, leff)
            bounds.append((lo, hi))
    return bounds


def _tree_max(terms):
    while len(terms) > 1:
        nxt = [jnp.maximum(a, b) for a, b in zip(terms[::2], terms[1::2])]
        if len(terms) % 2:
            nxt.append(terms[-1])
        terms = nxt
    return terms[0]


def _tc_body(lens_ref, x_ref, o1_ref, o2_ref, o3_ref, *, b0):
    i = pl.program_id(0)
    t = pl.program_id(1)
    T = x_ref.shape[2]
    L = pl.num_programs(1) * T

    leff = jnp.minimum(lens_ref[i + b0] - SHIFT, L)
    xb = x_ref[0]
    p0 = t * T
    p1 = p0 + T
    pos = lax.broadcasted_iota(jnp.int32, (1, T), 1) + p0

    @pl.when(t == 0)
    def _():
        o1_ref[...] = jnp.full_like(o1_ref, NEG_INF)
        o2_ref[...] = jnp.full_like(o2_ref, NEG_INF)
        o3_ref[...] = jnp.full_like(o3_ref, NEG_INF)

    full_m = jnp.max(xb, axis=1)

    def acc(o_ref, idx, lo, hi):
        inside = (lo <= p0) & (p1 <= hi)
        partial = (lo < p1) & (hi > p0) & jnp.logical_not(inside)

        @pl.when(inside)
        def _():
            o_ref[idx] = jnp.maximum(o_ref[idx], full_m)

        @pl.when(partial)
        def _():
            mask = (pos >= lo) & (pos < hi)
            o_ref[idx] = jnp.maximum(
                o_ref[idx], jnp.max(jnp.where(mask, xb, NEG_INF), axis=1))

    bounds = _window_bounds(leff)
    acc(o1_ref, (0, 0, slice(None)), *bounds[0])
    for j in range(2):
        acc(o2_ref, (0, slice(None), j), *bounds[1 + j])
    for j in range(4):
        acc(o3_ref, (0, slice(None), j), *bounds[3 + j])


def _tc_call(x, lens, b0):
    B, C, L = x.shape
    NB = B - b0
    T = 1024
    NT = L // T
    grid_spec = pltpu.PrefetchScalarGridSpec(
        num_scalar_prefetch=1,
        grid=(NB, NT),
        in_specs=[
            pl.BlockSpec((1, C, T), lambda i, t, lens: (i + b0, 0, t)),
        ],
        out_specs=[
            pl.BlockSpec((1, 1, C), lambda i, t, lens: (i, 0, 0)),
            pl.BlockSpec((1, C, 2), lambda i, t, lens: (i, 0, 0)),
            pl.BlockSpec((1, C, 4), lambda i, t, lens: (i, 0, 0)),
        ],
    )
    o1, o2, o3 = pl.pallas_call(
        functools.partial(_tc_body, b0=b0),
        grid_spec=grid_spec,
        out_shape=[
            jax.ShapeDtypeStruct((NB, 1, C), jnp.float32),
            jax.ShapeDtypeStruct((NB, C, 2), jnp.float32),
            jax.ShapeDtypeStruct((NB, C, 4), jnp.float32),
        ],
    )(lens, x)
    return jnp.concatenate(
        [o1.reshape(NB, C), o2.reshape(NB, 2 * C), o3.reshape(NB, 4 * C)],
        axis=1)


def _make_sc_kernel(B, C, L, interpret=False):
    ROWS = B * C
    NW = 32
    RPW = ROWS // NW
    NSPAN = L // 256
    NBLK = L // 64
    NC = 2
    SB = 8
    NSB = RPW // SB

    def body(x_hbm, lens_hbm, out_hbm, lens_v, sbuf0, sbuf1, sv_ref, sv64_ref,
             s64_ref, pi_ref, ix_ref, p64_ref, praw_ref, corr_ref, scr_ref,
             out_buf, sem0, sem1):
        wid = lax.axis_index("s") * NC + lax.axis_index("c")
        row0 = wid * RPW
        sample = row0 // C

        pltpu.sync_copy(lens_hbm, lens_v)
        iota = lax.broadcasted_iota(jnp.int32, (LANES,), 0)
        iota16x = iota * LANES
        lvec = plsc.load_gather(lens_v, [iota * 0 + sample])
        leff = jnp.minimum(lvec[0] - SHIFT, L)
        windows = _window_bounds(leff)
        k2 = (leff + 1) // 2

        def vpen(cond):
            return jnp.where(cond, 0.0, NEG_INF).astype(jnp.float32)

        span_base = iota * 256
        raw_bases = {}
        for w in (2, 3, 4, 5, 6):
            lo, hi = windows[w]
            pi_ref[pl.ds(w * LANES, LANES)] = vpen(
                (span_base >= lo) & (span_base + 256 <= hi))
            b0blk = (lo // 256) * 4
            b1blk = ((hi - 1) // 256) * 4
            idx = jnp.where(iota < 4, b0blk + iota,
                            jnp.where(iota < 8, b1blk + (iota - 4), 0))
            ix_ref[pl.ds(w * LANES, LANES)] = idx
            bvec = idx * 64
            p64_ref[pl.ds(w * LANES, LANES)] = vpen(
                (bvec >= lo) & (bvec + 64 <= hi))
            bb = ((lo // 64) * 64, ((hi - 1) // 64) * 64)
            raw_bases[w] = bb
            for e in range(2):
                for k in range(4):
                    pos = iota + bb[e] + k * LANES
                    praw_ref[pl.ds(((w * 2 + e) * 4 + k) * LANES, LANES)] = \
                        vpen((pos >= lo) & (pos < hi))
        e1 = k2 - 1
        cbase = (e1 // LANES) * LANES
        corr_ref[...] = vpen(iota + cbase == e1)

        def compute_row(buf, r, slot):
            for g in range(NSPAN):
                subs = []
                for q in range(4):
                    base = g * 256 + q * 64
                    vs = [buf[r, pl.ds(base + k * LANES, LANES)]
                          for k in range(4)]
                    m = _tree_max(vs)
                    sv64_ref[pl.ds((g * 4 + q) * LANES, LANES)] = m
                    subs.append(m)
                sv_ref[pl.ds(g * LANES, LANES)] = _tree_max(subs)

            sspan = _tree_max([plsc.load_gather(sv_ref, [iota16x + j])
                               for j in range(LANES)])
            for p in range(4):
                part = _tree_max(
                    [plsc.load_gather(sv64_ref, [p * 256 + iota16x + j])
                     for j in range(LANES)])
                s64_ref[pl.ds(p * LANES, LANES)] = part

            accs = [None] * NWIN
            for w in (2, 3, 4, 5, 6):
                idx = ix_ref[pl.ds(w * LANES, LANES)]
                terms = [
                    sspan + pi_ref[pl.ds(w * LANES, LANES)],
                    plsc.load_gather(s64_ref, [idx])
                    + p64_ref[pl.ds(w * LANES, LANES)],
                ]
                for e in range(2):
                    for k in range(4):
                        terms.append(
                            buf[r, pl.ds(raw_bases[w][e] + k * LANES, LANES)]
                            + praw_ref[pl.ds(
                                ((w * 2 + e) * 4 + k) * LANES, LANES)])
                accs[w] = _tree_max(terms)

            corr = buf[r, pl.ds(cbase, LANES)] + corr_ref[...]
            accs[1] = jnp.maximum(jnp.maximum(accs[3], accs[4]), corr)
            accs[0] = jnp.maximum(accs[1], accs[2])

            for w in range(NWIN):
                scr_ref[pl.ds(w * LANES, LANES)] = accs[w]
            gathered = [plsc.load_gather(scr_ref, [iota16x + j])
                        for j in range(LANES)]
            out_buf[pl.ds(slot * LANES, LANES)] = _tree_max(gathered)

        def sb_src(sbi):
            return x_hbm.at[pl.ds(row0 + sbi * SB, SB)]

        pltpu.async_copy(sb_src(0), sbuf0, sem0)
        pltpu.async_copy(sb_src(1), sbuf1, sem1)

        @pl.loop(0, NSB, step=2)
        def sb_loop(g):
            for b, (sbuf, sem) in enumerate(((sbuf0, sem0), (sbuf1, sem1))):
                sbi = g + b
                pltpu.make_async_copy(sb_src(0), sbuf, sem).wait()

                @pl.loop(0, SB)
                def row_loop(r):
                    compute_row(sbuf, r, sbi * SB + r)

                nxt = sbi + 2

                @pl.when(nxt < NSB)
                def _():
                    pltpu.async_copy(sb_src(nxt), sbuf, sem)

        pltpu.sync_copy(out_buf, out_hbm.at[pl.ds(row0 * LANES, RPW * LANES)])

    return pl.kernel(
        body,
        out_type=jax.ShapeDtypeStruct((ROWS * LANES,), jnp.float32),
        mesh=plsc.VectorSubcoreMesh(core_axis_name="c", subcore_axis_name="s",
                                    num_cores=NC, num_subcores=16),
        compiler_params=pltpu.CompilerParams(needs_layout_passes=False),
        scratch_types=[
            pltpu.VMEM((LANES,), jnp.int32),
            pltpu.VMEM((SB, L), jnp.float32),
            pltpu.VMEM((SB, L), jnp.float32),
            pltpu.VMEM((NSPAN * LANES,), jnp.float32),
            pltpu.VMEM((NBLK * LANES,), jnp.float32),
            pltpu.VMEM((NBLK,), jnp.float32),
            pltpu.VMEM((NWIN * LANES,), jnp.float32),
            pltpu.VMEM((NWIN * LANES,), jnp.int32),
            pltpu.VMEM((NWIN * LANES,), jnp.float32),
            pltpu.VMEM((NWIN * 8 * LANES,), jnp.float32),
            pltpu.VMEM((LANES,), jnp.float32),
            pltpu.VMEM((LANES * LANES,), jnp.float32),
            pltpu.VMEM((RPW * LANES,), jnp.float32),
            pltpu.SemaphoreType.DMA,
            pltpu.SemaphoreType.DMA,
        ],
        interpret=interpret,
    )


def kernel(x, orig_len):
    B, C, L = x.shape
    BSC = 4
    lens = jnp.asarray(orig_len, jnp.int32)
    sc = _make_sc_kernel(BSC, C, L)
    out = sc(x.reshape(B * C, L), lens).reshape(BSC, C, LANES)
    sc_rows = jnp.concatenate(
        [out[:, :, 0],
         out[:, :, 1:3].reshape(BSC, 2 * C),
         out[:, :, 3:7].reshape(BSC, 4 * C)], axis=1)
    tc_rows = _tc_call(x, lens, BSC)
    return jnp.concatenate([sc_rows, tc_rows], axis=0)
